# Initial kernel scaffold; baseline (speedup 1.0000x reference)
import jax
import jax.numpy as jnp
from jax.experimental import pallas as pl


def kernel(latent, latent_norm, dist_X, pair_mask_X):
    raise NotImplementedError("write your pallas kernel here")


# speedup vs baseline: 9.1263x; 9.1263x over previous
import functools

import jax
import jax.numpy as jnp
from jax.experimental import pallas as pl
from jax.experimental.pallas import tpu as pltpu

N = 1024
D = 64
K = 16
R = 256
GRID = N // R


def _knn_body(norm_ref, latb_ref, lat_ref, dx_ref, mx_ref,
              mz_ref, d12_ref, d21_ref):
    i = pl.program_id(0)

    x_blk = latb_ref[...]
    x_all = lat_ref[...]
    r_blk = jnp.sum(x_blk * x_blk, axis=1)
    r_all = jnp.sum(x_all * x_all, axis=1)

    g = jax.lax.dot_general(x_blk, x_all, (((1,), (1,)), ((), ())),
                            preferred_element_type=jnp.float32)
    sq = r_blk[:, None] + r_all[None, :] - 2.0 * g
    dist = jnp.sqrt(jnp.maximum(sq, 0.0))
    dist = dist * (1.0 / norm_ref[0, 0])

    rows = i * R + jax.lax.broadcasted_iota(jnp.int32, (R, N), 0)
    cols = jax.lax.broadcasted_iota(jnp.int32, (R, N), 1)
    dist = jnp.where(rows == cols, 0.0, dist)

    big = jnp.int32(N + 1)
    inf = jnp.float32(jnp.inf)
    d = dist
    mz = jnp.zeros((R, N), jnp.float32)
    for t in range(K + 1):
        m = jnp.min(d, axis=1, keepdims=True)
        cand = jnp.where(d == m, cols, big)
        jmin = jnp.min(cand, axis=1, keepdims=True)
        first = cols == jmin
        if t > 0:
            mz = jnp.where(first, 1.0, mz)
        d = jnp.where(first, inf, d)

    dx = dx_ref[...]
    mx = mx_ref[...]
    v1 = mx * dx - mx * dist
    s1 = jnp.sum(v1 * v1)
    v2 = mz * dx - mz * dist
    s2 = jnp.sum(v2 * v2)

    mz_ref[...] = mz

    @pl.when(i == 0)
    def _():
        d12_ref[0, 0] = 0.0
        d21_ref[0, 0] = 0.0

    d12_ref[0, 0] += s1
    d21_ref[0, 0] += s2


def _count_body(d12_ref, d21_ref, zr_ref, zc_ref, xr_ref, xc_ref,
                dist_ref, nm_ref, o12_ref, o21_ref, acc_ref):
    i = pl.program_id(0)

    zr = zr_ref[...]
    zc = zc_ref[...]
    wr = jnp.where(xr_ref[...] != 0.0, 0.0, 1.0)
    wc = jnp.where(xc_ref[...] != 0.0, 0.0, 1.0)
    mr = zr * wr
    mc = zc * wc

    dn = (((1,), (0,)), ((), ()))
    p_aa = jax.lax.dot_general(zr, zc, dn, preferred_element_type=jnp.float32)
    p_mw = jax.lax.dot_general(mr, wc, dn, preferred_element_type=jnp.float32)
    p_mm = jax.lax.dot_general(mr, mc, dn, preferred_element_type=jnp.float32)

    ii = jax.lax.broadcasted_iota(jnp.int32, (R, R), 0)
    jj = jax.lax.broadcasted_iota(jnp.int32, (R, R), 1)
    eye = ii == jj
    t_aa = jnp.sum(jnp.where(eye, p_aa, 0.0))
    t_mw = jnp.sum(jnp.where(eye, p_mw, 0.0))
    t_mm = jnp.sum(jnp.where(eye, p_mm, 0.0))
    s_a = jnp.sum(zr)

    @pl.when(i == 0)
    def _():
        acc_ref[0] = 0.0
        acc_ref[1] = 0.0
        acc_ref[2] = 0.0
        acc_ref[3] = 0.0

    acc_ref[0] += s_a
    acc_ref[1] += t_aa
    acc_ref[2] += t_mw
    acc_ref[3] += t_mm

    @pl.when(i == GRID - 1)
    def _():
        zsum = 2.0 * acc_ref[0] - acc_ref[1]
        nm = 2.0 * acc_ref[2] - acc_ref[3]
        nm_ref[0, 0] = nm / zsum
        d12 = d12_ref[0, 0]
        d21 = d21_ref[0, 0]
        o12_ref[0, 0] = d12
        o21_ref[0, 0] = d21
        dist_ref[0, 0] = d12 + d21


@jax.jit
def kernel(latent, latent_norm, dist_X, pair_mask_X):
    norm2d = latent_norm.reshape(1, 1)

    mask_z, d12, d21 = pl.pallas_call(
        _knn_body,
        grid=(GRID,),
        in_specs=[
            pl.BlockSpec(memory_space=pltpu.SMEM),
            pl.BlockSpec((R, D), lambda i: (i, 0)),
            pl.BlockSpec((N, D), lambda i: (0, 0)),
            pl.BlockSpec((R, N), lambda i: (i, 0)),
            pl.BlockSpec((R, N), lambda i: (i, 0)),
        ],
        out_specs=[
            pl.BlockSpec((R, N), lambda i: (i, 0)),
            pl.BlockSpec(memory_space=pltpu.SMEM),
            pl.BlockSpec(memory_space=pltpu.SMEM),
        ],
        out_shape=[
            jax.ShapeDtypeStruct((N, N), jnp.float32),
            jax.ShapeDtypeStruct((1, 1), jnp.float32),
            jax.ShapeDtypeStruct((1, 1), jnp.float32),
        ],
    )(norm2d, latent, latent, dist_X, pair_mask_X)

    distance, nm, o12, o21 = pl.pallas_call(
        _count_body,
        grid=(GRID,),
        in_specs=[
            pl.BlockSpec(memory_space=pltpu.SMEM),
            pl.BlockSpec(memory_space=pltpu.SMEM),
            pl.BlockSpec((R, N), lambda i: (i, 0)),
            pl.BlockSpec((N, R), lambda i: (0, i)),
            pl.BlockSpec((R, N), lambda i: (i, 0)),
            pl.BlockSpec((N, R), lambda i: (0, i)),
        ],
        out_specs=[
            pl.BlockSpec(memory_space=pltpu.SMEM),
            pl.BlockSpec(memory_space=pltpu.SMEM),
            pl.BlockSpec(memory_space=pltpu.SMEM),
            pl.BlockSpec(memory_space=pltpu.SMEM),
        ],
        out_shape=[
            jax.ShapeDtypeStruct((1, 1), jnp.float32),
            jax.ShapeDtypeStruct((1, 1), jnp.float32),
            jax.ShapeDtypeStruct((1, 1), jnp.float32),
            jax.ShapeDtypeStruct((1, 1), jnp.float32),
        ],
        scratch_shapes=[pltpu.SMEM((4,), jnp.float32)],
    )(d12, d21, mask_z, mask_z, pair_mask_X, pair_mask_X)

    return (distance.reshape(()), nm.reshape(()),
            o12.reshape(()), o21.reshape(()))


# speedup vs baseline: 12.2417x; 1.3414x over previous
import functools

import jax
import jax.numpy as jnp
from jax.experimental import pallas as pl
from jax.experimental.pallas import tpu as pltpu

N = 1024
D = 64
K = 16
R = 256
GRID = N // R


def _knn_body(norm_ref, latb_ref, lat_ref, dx_ref, mx_ref,
              mz_ref, d12_ref, d21_ref):
    i = pl.program_id(0)

    x_blk = latb_ref[...]
    x_all = lat_ref[...]
    r_blk = jnp.sum(x_blk * x_blk, axis=1)
    r_all = jnp.sum(x_all * x_all, axis=1)

    g = jax.lax.dot_general(x_blk, x_all, (((1,), (1,)), ((), ())),
                            preferred_element_type=jnp.float32)
    sq = r_blk[:, None] + r_all[None, :] - 2.0 * g
    dist = jnp.sqrt(jnp.maximum(sq, 0.0))
    dist = dist * (1.0 / norm_ref[0, 0])

    rows = i * R + jax.lax.broadcasted_iota(jnp.int32, (R, N), 0)
    cols = jax.lax.broadcasted_iota(jnp.int32, (R, N), 1)
    dist = jnp.where(rows == cols, 0.0, dist)

    inf = jnp.float32(jnp.inf)
    kbits = jax.lax.bitcast_convert_type(dist, jnp.int32)
    kbits = jnp.bitwise_or(jnp.bitwise_and(kbits, jnp.int32(~1023)), cols)
    kf = jax.lax.bitcast_convert_type(kbits, jnp.float32)
    mz = jnp.zeros((R, N), jnp.float32)
    for t in range(K + 1):
        m = jnp.min(kf, axis=1, keepdims=True)
        first = kf == m
        if t > 0:
            mz = jnp.where(first, 1.0, mz)
        kf = jnp.where(first, inf, kf)

    dx = dx_ref[...]
    mx = mx_ref[...]
    v1 = mx * dx - mx * dist
    s1 = jnp.sum(v1 * v1)
    v2 = mz * dx - mz * dist
    s2 = jnp.sum(v2 * v2)

    mz_ref[...] = mz

    @pl.when(i == 0)
    def _():
        d12_ref[0, 0] = 0.0
        d21_ref[0, 0] = 0.0

    d12_ref[0, 0] += s1
    d21_ref[0, 0] += s2


def _count_body(d12_ref, d21_ref, zr_ref, zc_ref, xr_ref, xc_ref,
                dist_ref, nm_ref, o12_ref, o21_ref, acc_ref):
    i = pl.program_id(0)

    zr = zr_ref[...]
    zc = zc_ref[...]
    wr = jnp.where(xr_ref[...] != 0.0, 0.0, 1.0)
    wc = jnp.where(xc_ref[...] != 0.0, 0.0, 1.0)
    mr = zr * wr
    mc = zc * wc

    dn = (((1,), (0,)), ((), ()))
    p_aa = jax.lax.dot_general(zr, zc, dn, preferred_element_type=jnp.float32)
    p_mw = jax.lax.dot_general(mr, wc, dn, preferred_element_type=jnp.float32)
    p_mm = jax.lax.dot_general(mr, mc, dn, preferred_element_type=jnp.float32)

    ii = jax.lax.broadcasted_iota(jnp.int32, (R, R), 0)
    jj = jax.lax.broadcasted_iota(jnp.int32, (R, R), 1)
    eye = ii == jj
    t_aa = jnp.sum(jnp.where(eye, p_aa, 0.0))
    t_mw = jnp.sum(jnp.where(eye, p_mw, 0.0))
    t_mm = jnp.sum(jnp.where(eye, p_mm, 0.0))
    s_a = jnp.sum(zr)

    @pl.when(i == 0)
    def _():
        acc_ref[0] = 0.0
        acc_ref[1] = 0.0
        acc_ref[2] = 0.0
        acc_ref[3] = 0.0

    acc_ref[0] += s_a
    acc_ref[1] += t_aa
    acc_ref[2] += t_mw
    acc_ref[3] += t_mm

    @pl.when(i == GRID - 1)
    def _():
        zsum = 2.0 * acc_ref[0] - acc_ref[1]
        nm = 2.0 * acc_ref[2] - acc_ref[3]
        nm_ref[0, 0] = nm / zsum
        d12 = d12_ref[0, 0]
        d21 = d21_ref[0, 0]
        o12_ref[0, 0] = d12
        o21_ref[0, 0] = d21
        dist_ref[0, 0] = d12 + d21


@jax.jit
def kernel(latent, latent_norm, dist_X, pair_mask_X):
    norm2d = latent_norm.reshape(1, 1)

    mask_z, d12, d21 = pl.pallas_call(
        _knn_body,
        grid=(GRID,),
        in_specs=[
            pl.BlockSpec(memory_space=pltpu.SMEM),
            pl.BlockSpec((R, D), lambda i: (i, 0)),
            pl.BlockSpec((N, D), lambda i: (0, 0)),
            pl.BlockSpec((R, N), lambda i: (i, 0)),
            pl.BlockSpec((R, N), lambda i: (i, 0)),
        ],
        out_specs=[
            pl.BlockSpec((R, N), lambda i: (i, 0)),
            pl.BlockSpec(memory_space=pltpu.SMEM),
            pl.BlockSpec(memory_space=pltpu.SMEM),
        ],
        out_shape=[
            jax.ShapeDtypeStruct((N, N), jnp.float32),
            jax.ShapeDtypeStruct((1, 1), jnp.float32),
            jax.ShapeDtypeStruct((1, 1), jnp.float32),
        ],
    )(norm2d, latent, latent, dist_X, pair_mask_X)

    distance, nm, o12, o21 = pl.pallas_call(
        _count_body,
        grid=(GRID,),
        in_specs=[
            pl.BlockSpec(memory_space=pltpu.SMEM),
            pl.BlockSpec(memory_space=pltpu.SMEM),
            pl.BlockSpec((R, N), lambda i: (i, 0)),
            pl.BlockSpec((N, R), lambda i: (0, i)),
            pl.BlockSpec((R, N), lambda i: (i, 0)),
            pl.BlockSpec((N, R), lambda i: (0, i)),
        ],
        out_specs=[
            pl.BlockSpec(memory_space=pltpu.SMEM),
            pl.BlockSpec(memory_space=pltpu.SMEM),
            pl.BlockSpec(memory_space=pltpu.SMEM),
            pl.BlockSpec(memory_space=pltpu.SMEM),
        ],
        out_shape=[
            jax.ShapeDtypeStruct((1, 1), jnp.float32),
            jax.ShapeDtypeStruct((1, 1), jnp.float32),
            jax.ShapeDtypeStruct((1, 1), jnp.float32),
            jax.ShapeDtypeStruct((1, 1), jnp.float32),
        ],
        scratch_shapes=[pltpu.SMEM((4,), jnp.float32)],
    )(d12, d21, mask_z, mask_z, pair_mask_X, pair_mask_X)

    return (distance.reshape(()), nm.reshape(()),
            o12.reshape(()), o21.reshape(()))


# speedup vs baseline: 15.7440x; 1.2861x over previous
import jax
import jax.numpy as jnp
from jax.experimental import pallas as pl
from jax.experimental.pallas import tpu as pltpu

N = 1024
D = 64
K = 16
R = 256
GRID = N // R


def _dist_and_keys(norm_ref, latb_ref, lat_ref, i):
    x_blk = latb_ref[...]
    x_all = lat_ref[...]
    r_blk = jnp.sum(x_blk * x_blk, axis=1)
    r_all = jnp.sum(x_all * x_all, axis=1)

    g = jax.lax.dot_general(x_blk, x_all, (((1,), (1,)), ((), ())),
                            preferred_element_type=jnp.float32)
    sq = r_blk[:, None] + r_all[None, :] - 2.0 * g
    dist = jnp.sqrt(jnp.maximum(sq, 0.0))
    dist = dist * (1.0 / norm_ref[0, 0])

    rows = i * R + jax.lax.broadcasted_iota(jnp.int32, (R, N), 0)
    cols = jax.lax.broadcasted_iota(jnp.int32, (R, N), 1)
    diag = rows == cols
    dist = jnp.where(diag, 0.0, dist)

    banded = jnp.bitwise_and(jax.lax.bitcast_convert_type(dist, jnp.int32),
                             jnp.int32(~1023))
    inf = jnp.float32(jnp.inf)
    kf_c = jax.lax.bitcast_convert_type(jnp.bitwise_or(banded, cols),
                                        jnp.float32)
    kf_c = jnp.where(diag, inf, kf_c)
    kf_r = jax.lax.bitcast_convert_type(jnp.bitwise_or(banded, rows),
                                        jnp.float32)
    kf_r = jnp.where(diag, inf, kf_r)
    return dist, kf_c, kf_r


def _body(norm_ref, latb_ref, lat_ref, dx_ref, mx_ref, mxc_ref,
          dist_ref, nm_ref, o12_ref, o21_ref,
          tcol_ref, trow_ref, acc_ref):
    p = pl.program_id(0)
    i = pl.program_id(1)
    inf = jnp.float32(jnp.inf)

    @pl.when(p == 0)
    def _phase_a():
        _, kf, _ = _dist_and_keys(norm_ref, latb_ref, lat_ref, i)
        for t in range(K):
            m = jnp.min(kf, axis=1, keepdims=True)
            if t < K - 1:
                kf = jnp.where(kf == m, inf, kf)
        tcol_ref[pl.ds(i * R, R), :] = m
        ii = jax.lax.broadcasted_iota(jnp.int32, (R, R), 0)
        jj = jax.lax.broadcasted_iota(jnp.int32, (R, R), 1)
        eye = jnp.where(ii == jj, 1.0, 0.0)
        mrow = jax.lax.dot_general(m, eye, (((0,), (0,)), ((), ())),
                                   preferred_element_type=jnp.float32)
        trow_ref[:, pl.ds(i * R, R)] = mrow

        @pl.when(i == 0)
        def _():
            acc_ref[0] = 0.0
            acc_ref[1] = 0.0
            acc_ref[2] = 0.0
            acc_ref[3] = 0.0

    @pl.when(p == 1)
    def _phase_b():
        dist, kf_c, kf_r = _dist_and_keys(norm_ref, latb_ref, lat_ref, i)
        tcol = tcol_ref[pl.ds(i * R, R), :]
        trow = trow_ref[...]

        zr = kf_c <= tcol
        zc = kf_r <= trow
        ztot = jnp.where(jnp.logical_or(zr, zc), 1.0, 0.0)

        dx = dx_ref[...]
        mx = mx_ref[...]
        v1 = mx * dx - mx * dist
        s1 = jnp.sum(v1 * v1)
        dd = dx - dist
        v2 = jnp.where(zr, dd, 0.0)
        s2 = jnp.sum(v2 * v2)

        c = jnp.where(mx != 0.0, 1.0, 0.0)
        zc_op = ztot * c
        cc = jnp.where(mxc_ref[...] != 0.0, 1.0, 0.0)
        prod = jax.lax.dot_general(
            zc_op.astype(jnp.bfloat16), cc.astype(jnp.bfloat16),
            (((1,), (0,)), ((), ())), preferred_element_type=jnp.float32)
        ii = jax.lax.broadcasted_iota(jnp.int32, (R, R), 0)
        jj = jax.lax.broadcasted_iota(jnp.int32, (R, R), 1)
        t_zcc = jnp.sum(jnp.where(ii == jj, prod, 0.0))

        acc_ref[0] += s1
        acc_ref[1] += s2
        acc_ref[2] += jnp.sum(ztot)
        acc_ref[3] += 2.0 * jnp.sum(zc_op) - t_zcc

        @pl.when(i == GRID - 1)
        def _():
            s1t = acc_ref[0]
            s2t = acc_ref[1]
            o12_ref[0, 0] = s1t
            o21_ref[0, 0] = s2t
            dist_ref[0, 0] = s1t + s2t
            nm_ref[0, 0] = (acc_ref[2] - acc_ref[3]) / acc_ref[2]


@jax.jit
def kernel(latent, latent_norm, dist_X, pair_mask_X):
    norm2d = latent_norm.reshape(1, 1)

    distance, nm, o12, o21 = pl.pallas_call(
        _body,
        grid=(2, GRID),
        in_specs=[
            pl.BlockSpec(memory_space=pltpu.SMEM),
            pl.BlockSpec((R, D), lambda p, i: (i, 0)),
            pl.BlockSpec((N, D), lambda p, i: (0, 0)),
            pl.BlockSpec((R, N), lambda p, i: (i * p, 0)),
            pl.BlockSpec((R, N), lambda p, i: (i * p, 0)),
            pl.BlockSpec((N, R), lambda p, i: (0, i * p)),
        ],
        out_specs=[
            pl.BlockSpec(memory_space=pltpu.SMEM),
            pl.BlockSpec(memory_space=pltpu.SMEM),
            pl.BlockSpec(memory_space=pltpu.SMEM),
            pl.BlockSpec(memory_space=pltpu.SMEM),
        ],
        out_shape=[
            jax.ShapeDtypeStruct((1, 1), jnp.float32),
            jax.ShapeDtypeStruct((1, 1), jnp.float32),
            jax.ShapeDtypeStruct((1, 1), jnp.float32),
            jax.ShapeDtypeStruct((1, 1), jnp.float32),
        ],
        scratch_shapes=[
            pltpu.VMEM((N, 1), jnp.float32),
            pltpu.VMEM((1, N), jnp.float32),
            pltpu.SMEM((4,), jnp.float32),
        ],
    )(norm2d, latent, latent, dist_X, pair_mask_X, pair_mask_X)

    return (distance.reshape(()), nm.reshape(()),
            o12.reshape(()), o21.reshape(()))


# speedup vs baseline: 16.2858x; 1.0344x over previous
import jax
import jax.numpy as jnp
from jax.experimental import pallas as pl
from jax.experimental.pallas import tpu as pltpu

N = 1024
D = 64
K = 16
R = 256
GRID = N // R


def _dist_and_keys(norm_ref, latb_ref, lat_ref, i):
    x_blk = latb_ref[...]
    x_all = lat_ref[...]
    r_blk = jnp.sum(x_blk * x_blk, axis=1)
    r_all = jnp.sum(x_all * x_all, axis=1)

    g = jax.lax.dot_general(x_blk, x_all, (((1,), (1,)), ((), ())),
                            preferred_element_type=jnp.float32)
    sq = r_blk[:, None] + r_all[None, :] - 2.0 * g
    dist = jnp.sqrt(jnp.maximum(sq, 0.0))
    dist = dist * (1.0 / norm_ref[0, 0])

    rows = i * R + jax.lax.broadcasted_iota(jnp.int32, (R, N), 0)
    cols = jax.lax.broadcasted_iota(jnp.int32, (R, N), 1)
    diag = rows == cols
    dist = jnp.where(diag, 0.0, dist)

    kf = jnp.where(diag, jnp.float32(jnp.inf), dist)
    return dist, kf


def _body(norm_ref, latb_ref, lat_ref, dx_ref, mx_ref, mxc_ref,
          dist_ref, nm_ref, o12_ref, o21_ref,
          tcol_ref, trow_ref, acc_ref):
    p = pl.program_id(0)
    i = pl.program_id(1)
    inf = jnp.float32(jnp.inf)

    @pl.when(p == 0)
    def _phase_a():
        _, kf = _dist_and_keys(norm_ref, latb_ref, lat_ref, i)
        for t in range(K):
            m = jnp.min(kf, axis=1, keepdims=True)
            if t < K - 1:
                kf = jnp.where(kf == m, inf, kf)
        tcol_ref[pl.ds(i * R, R), :] = m
        ii = jax.lax.broadcasted_iota(jnp.int32, (R, R), 0)
        jj = jax.lax.broadcasted_iota(jnp.int32, (R, R), 1)
        eye = jnp.where(ii == jj, 1.0, 0.0)
        mrow = jax.lax.dot_general(m, eye, (((0,), (0,)), ((), ())),
                                   preferred_element_type=jnp.float32)
        trow_ref[:, pl.ds(i * R, R)] = mrow

        @pl.when(i == 0)
        def _():
            acc_ref[0] = 0.0
            acc_ref[1] = 0.0
            acc_ref[2] = 0.0
            acc_ref[3] = 0.0

    @pl.when(p == 1)
    def _phase_b():
        dist, kf = _dist_and_keys(norm_ref, latb_ref, lat_ref, i)
        tcol = tcol_ref[pl.ds(i * R, R), :]
        trow = trow_ref[...]

        zr = kf <= tcol
        zc = kf <= trow
        ztot = jnp.where(jnp.logical_or(zr, zc), 1.0, 0.0)

        dx = dx_ref[...]
        mx = mx_ref[...]
        v1 = mx * dx - mx * dist
        s1 = jnp.sum(v1 * v1)
        dd = dx - dist
        v2 = jnp.where(zr, dd, 0.0)
        s2 = jnp.sum(v2 * v2)

        c = jnp.where(mx != 0.0, 1.0, 0.0)
        zc_op = ztot * c
        cc = jnp.where(mxc_ref[...] != 0.0, 1.0, 0.0)
        prod = jax.lax.dot_general(
            zc_op.astype(jnp.bfloat16), cc.astype(jnp.bfloat16),
            (((1,), (0,)), ((), ())), preferred_element_type=jnp.float32)
        ii = jax.lax.broadcasted_iota(jnp.int32, (R, R), 0)
        jj = jax.lax.broadcasted_iota(jnp.int32, (R, R), 1)
        t_zcc = jnp.sum(jnp.where(ii == jj, prod, 0.0))

        acc_ref[0] += s1
        acc_ref[1] += s2
        acc_ref[2] += jnp.sum(ztot)
        acc_ref[3] += 2.0 * jnp.sum(zc_op) - t_zcc

        @pl.when(i == GRID - 1)
        def _():
            s1t = acc_ref[0]
            s2t = acc_ref[1]
            o12_ref[0, 0] = s1t
            o21_ref[0, 0] = s2t
            dist_ref[0, 0] = s1t + s2t
            nm_ref[0, 0] = (acc_ref[2] - acc_ref[3]) / acc_ref[2]


@jax.jit
def kernel(latent, latent_norm, dist_X, pair_mask_X):
    norm2d = latent_norm.reshape(1, 1)

    distance, nm, o12, o21 = pl.pallas_call(
        _body,
        grid=(2, GRID),
        in_specs=[
            pl.BlockSpec(memory_space=pltpu.SMEM),
            pl.BlockSpec((R, D), lambda p, i: (i, 0)),
            pl.BlockSpec((N, D), lambda p, i: (0, 0)),
            pl.BlockSpec((R, N), lambda p, i: (i * p, 0)),
            pl.BlockSpec((R, N), lambda p, i: (i * p, 0)),
            pl.BlockSpec((N, R), lambda p, i: (0, i * p)),
        ],
        out_specs=[
            pl.BlockSpec(memory_space=pltpu.SMEM),
            pl.BlockSpec(memory_space=pltpu.SMEM),
            pl.BlockSpec(memory_space=pltpu.SMEM),
            pl.BlockSpec(memory_space=pltpu.SMEM),
        ],
        out_shape=[
            jax.ShapeDtypeStruct((1, 1), jnp.float32),
            jax.ShapeDtypeStruct((1, 1), jnp.float32),
            jax.ShapeDtypeStruct((1, 1), jnp.float32),
            jax.ShapeDtypeStruct((1, 1), jnp.float32),
        ],
        scratch_shapes=[
            pltpu.VMEM((N, 1), jnp.float32),
            pltpu.VMEM((1, N), jnp.float32),
            pltpu.SMEM((4,), jnp.float32),
        ],
    )(norm2d, latent, latent, dist_X, pair_mask_X, pair_mask_X)

    return (distance.reshape(()), nm.reshape(()),
            o12.reshape(()), o21.reshape(()))


# speedup vs baseline: 16.8382x; 1.0339x over previous
import jax
import jax.numpy as jnp
from jax.experimental import pallas as pl
from jax.experimental.pallas import tpu as pltpu

N = 1024
D = 64
K = 16
R = 256
GRID = N // R


def _dist_and_keys(norm_ref, af_ref, bf_ref, i):
    a_blk = af_ref[pl.ds(i * R, R), :]
    b_all = bf_ref[...]
    sq = jax.lax.dot_general(a_blk, b_all, (((1,), (1,)), ((), ())),
                             preferred_element_type=jnp.float32)
    dist = jnp.sqrt(jnp.maximum(sq, 0.0))
    dist = dist * (1.0 / norm_ref[0, 0])

    rows = i * R + jax.lax.broadcasted_iota(jnp.int32, (R, N), 0)
    cols = jax.lax.broadcasted_iota(jnp.int32, (R, N), 1)
    diag = rows == cols
    dist = jnp.where(diag, 0.0, dist)

    kf = jnp.where(diag, jnp.float32(jnp.inf), dist)
    return dist, kf


def _body(norm_ref, lat_ref, dx_ref, mx_ref, mxc_ref,
          dist_ref, nm_ref, o12_ref, o21_ref,
          af_ref, bf_ref, tcol_ref, trow_ref, acc_ref):
    p = pl.program_id(0)
    i = pl.program_id(1)
    inf = jnp.float32(jnp.inf)

    @pl.when(jnp.logical_and(p == 0, i == 0))
    def _build_augmented():
        x = lat_ref[...]
        r = jnp.sum(x * x, axis=1)[:, None]
        one = jnp.ones((N, 1), jnp.float32)
        af_ref[:, pl.ds(0, D)] = x
        af_ref[:, pl.ds(D, 1)] = r
        af_ref[:, pl.ds(D + 1, 1)] = one
        bf_ref[:, pl.ds(0, D)] = -2.0 * x
        bf_ref[:, pl.ds(D, 1)] = one
        bf_ref[:, pl.ds(D + 1, 1)] = r

    @pl.when(p == 0)
    def _phase_a():
        _, kf = _dist_and_keys(norm_ref, af_ref, bf_ref, i)
        for t in range(K):
            m = jnp.min(kf, axis=1, keepdims=True)
            if t < K - 1:
                kf = jnp.where(kf == m, inf, kf)
        tcol_ref[pl.ds(i * R, R), :] = m
        ii = jax.lax.broadcasted_iota(jnp.int32, (R, R), 0)
        jj = jax.lax.broadcasted_iota(jnp.int32, (R, R), 1)
        eye = jnp.where(ii == jj, 1.0, 0.0)
        mrow = jax.lax.dot_general(m, eye, (((0,), (0,)), ((), ())),
                                   preferred_element_type=jnp.float32)
        trow_ref[:, pl.ds(i * R, R)] = mrow

        @pl.when(i == 0)
        def _():
            acc_ref[0] = 0.0
            acc_ref[1] = 0.0
            acc_ref[2] = 0.0
            acc_ref[3] = 0.0

    @pl.when(p == 1)
    def _phase_b():
        dist, kf = _dist_and_keys(norm_ref, af_ref, bf_ref, i)
        tcol = tcol_ref[pl.ds(i * R, R), :]
        trow = trow_ref[...]

        zr = kf <= tcol
        zc = kf <= trow
        ztot = jnp.where(jnp.logical_or(zr, zc), 1.0, 0.0)

        dx = dx_ref[...]
        mx = mx_ref[...]
        v1 = mx * dx - mx * dist
        s1 = jnp.sum(v1 * v1)
        dd = dx - dist
        v2 = jnp.where(zr, dd, 0.0)
        s2 = jnp.sum(v2 * v2)

        c = jnp.where(mx != 0.0, 1.0, 0.0)
        zc_op = ztot * c
        cc = jnp.where(mxc_ref[...] != 0.0, 1.0, 0.0)
        prod = jax.lax.dot_general(
            zc_op.astype(jnp.bfloat16), cc.astype(jnp.bfloat16),
            (((1,), (0,)), ((), ())), preferred_element_type=jnp.float32)
        ii = jax.lax.broadcasted_iota(jnp.int32, (R, R), 0)
        jj = jax.lax.broadcasted_iota(jnp.int32, (R, R), 1)
        t_zcc = jnp.sum(jnp.where(ii == jj, prod, 0.0))

        acc_ref[0] += s1
        acc_ref[1] += s2
        acc_ref[2] += jnp.sum(ztot)
        acc_ref[3] += 2.0 * jnp.sum(zc_op) - t_zcc

        @pl.when(i == GRID - 1)
        def _():
            s1t = acc_ref[0]
            s2t = acc_ref[1]
            o12_ref[0, 0] = s1t
            o21_ref[0, 0] = s2t
            dist_ref[0, 0] = s1t + s2t
            nm_ref[0, 0] = (acc_ref[2] - acc_ref[3]) / acc_ref[2]


@jax.jit
def kernel(latent, latent_norm, dist_X, pair_mask_X):
    norm2d = latent_norm.reshape(1, 1)

    distance, nm, o12, o21 = pl.pallas_call(
        _body,
        grid=(2, GRID),
        in_specs=[
            pl.BlockSpec(memory_space=pltpu.SMEM),
            pl.BlockSpec((N, D), lambda p, i: (0, 0)),
            pl.BlockSpec((R, N), lambda p, i: (i * p, 0)),
            pl.BlockSpec((R, N), lambda p, i: (i * p, 0)),
            pl.BlockSpec((N, R), lambda p, i: (0, i * p)),
        ],
        out_specs=[
            pl.BlockSpec(memory_space=pltpu.SMEM),
            pl.BlockSpec(memory_space=pltpu.SMEM),
            pl.BlockSpec(memory_space=pltpu.SMEM),
            pl.BlockSpec(memory_space=pltpu.SMEM),
        ],
        out_shape=[
            jax.ShapeDtypeStruct((1, 1), jnp.float32),
            jax.ShapeDtypeStruct((1, 1), jnp.float32),
            jax.ShapeDtypeStruct((1, 1), jnp.float32),
            jax.ShapeDtypeStruct((1, 1), jnp.float32),
        ],
        scratch_shapes=[
            pltpu.VMEM((N, D + 2), jnp.float32),
            pltpu.VMEM((N, D + 2), jnp.float32),
            pltpu.VMEM((N, 1), jnp.float32),
            pltpu.VMEM((1, N), jnp.float32),
            pltpu.SMEM((4,), jnp.float32),
        ],
    )(norm2d, latent, dist_X, pair_mask_X, pair_mask_X)

    return (distance.reshape(()), nm.reshape(()),
            o12.reshape(()), o21.reshape(()))
